# baseline (speedup 1.0000x reference)
import functools

import jax
import jax.numpy as jnp
from jax import lax
from jax.experimental import pallas as pl
from jax.experimental.pallas import tpu as pltpu
from jax.experimental.pallas import tpu_sc as plsc

N = 10000
F = 256
H = 256
L = 128
E = 160000

R = 10240
EP = 163840
EDGES_PER_TILE = EP // 16
BSZ = 128
BLOCKS_PER_TILE = EDGES_PER_TILE // BSZ
DEG_CHUNK = EP // 32
ROWS_PER_TILE = R // 16
BM = 1024
NB = R // BM
NBUF = 2


def _mesh():
    return plsc.VectorSubcoreMesh(core_axis_name="c", subcore_axis_name="s")


def _deg_body(dsts_hbm, out_hbm, dst_v, hist_v, tmp_v, acc_v, shared):
    c = lax.axis_index("c")
    s = lax.axis_index("s")
    w = c * 16 + s
    z16 = jnp.zeros((16,), jnp.float32)

    def zero_hist(i, _):
        hist_v[pl.ds(i * 16, 16)] = z16
        return 0
    lax.fori_loop(0, R // 16, zero_hist, 0)

    pltpu.sync_copy(dsts_hbm.at[pl.ds(w * DEG_CHUNK, DEG_CHUNK)], dst_v)
    ones = jnp.ones((16,), jnp.float32)

    def hist_step(i, _):
        idx = dst_v[pl.ds(i * 16, 16)]
        plsc.addupdate_scatter(hist_v, [idx], ones)
        return 0
    lax.fori_loop(0, DEG_CHUNK // 16, hist_step, 0)

    pltpu.sync_copy(hist_v, shared.at[s])
    plsc.subcore_barrier()

    def zero_acc(i, _):
        acc_v[pl.ds(i * 16, 16)] = z16
        return 0
    lax.fori_loop(0, ROWS_PER_TILE // 16, zero_acc, 0)
    for t in range(16):
        pltpu.sync_copy(shared.at[t, pl.ds(s * ROWS_PER_TILE, ROWS_PER_TILE)], tmp_v)

        def add_step(j, _):
            acc_v[pl.ds(j * 16, 16)] = acc_v[pl.ds(j * 16, 16)] + tmp_v[pl.ds(j * 16, 16)]
            return 0
        lax.fori_loop(0, ROWS_PER_TILE // 16, add_step, 0)
    pltpu.sync_copy(acc_v, out_hbm.at[c, pl.ds(s * ROWS_PER_TILE, ROWS_PER_TILE)])


def _sc_degree(dsts):
    f = functools.partial(
        pl.kernel,
        out_type=jax.ShapeDtypeStruct((2, R), jnp.float32),
        mesh=_mesh(),
        scratch_types=[
            pltpu.VMEM((DEG_CHUNK,), jnp.int32),
            pltpu.VMEM((R,), jnp.float32),
            pltpu.VMEM((ROWS_PER_TILE,), jnp.float32),
            pltpu.VMEM((ROWS_PER_TILE,), jnp.float32),
            pltpu.VMEM_SHARED((16, R), jnp.float32),
        ],
        compiler_params=pltpu.CompilerParams(needs_layout_passes=False),
    )(_deg_body)
    return f(dsts)


WPR = L // 2

def _prop_body(table_hbm, srcs_hbm, dsts_hbm, zeros_hbm, out_hbm,
               sidx, dst_all, words, rowsf, accum, *sems):
    gsems = sems[:NBUF]
    ssems = sems[NBUF:2 * NBUF]
    isems = sems[2 * NBUF:]
    c = lax.axis_index("c")
    s = lax.axis_index("s")
    row0 = s * ROWS_PER_TILE
    pltpu.sync_copy(zeros_hbm.at[pl.ds(row0, ROWS_PER_TILE)],
                    accum.at[pl.ds(row0, ROWS_PER_TILE)])
    pltpu.sync_copy(dsts_hbm.at[s], dst_all)
    for b in range(NBUF):
        pltpu.sync_copy(srcs_hbm.at[c, s, b], sidx.at[b])
    plsc.subcore_barrier()

    def gather(b):
        return pltpu.async_copy(table_hbm.at[sidx.at[b]], words.at[b],
                                gsems[b])

    def scatter(k):
        return pltpu.async_copy(rowsf, accum.at[dst_all.at[k]],
                                ssems[0], add=True)

    mask_hi = jnp.full((16,), -65536, jnp.int32)

    def widen(b):
        def step(r0, _):
            for dr in range(8):
                r = r0 * 8 + dr
                for wv in range(WPR // 16):
                    w = words[b, r, pl.ds(wv * 16, 16)]
                    lo = plsc.bitcast(w << 16, jnp.float32)
                    hi = plsc.bitcast(w & mask_hi, jnp.float32)
                    rowsf[r, pl.ds(wv * 32, 16)] = lo
                    rowsf[r, pl.ds(wv * 32 + 16, 16)] = hi
            return 0
        lax.fori_loop(0, BSZ // 8, step, 0)

    for b in range(NBUF):
        gather(b)

    def group(k0, _):
        not_last = k0 < BLOCKS_PER_TILE // NBUF - 1
        for b in range(NBUF):
            k = k0 * NBUF + b
            pltpu.make_async_copy(table_hbm.at[sidx.at[b]], words.at[b],
                                  gsems[b]).wait()

            @pl.when(not_last)
            def _():
                pltpu.async_copy(srcs_hbm.at[c, s, k + NBUF], sidx.at[b],
                                 isems[b])

            @pl.when(k > 0)
            def _():
                pltpu.make_async_copy(rowsf, accum.at[dst_all.at[k]],
                                      ssems[0]).wait()
            widen(b)
            scatter(k)

            @pl.when(not_last)
            def _():
                pltpu.make_async_copy(srcs_hbm.at[c, s, k + NBUF], sidx.at[b],
                                      isems[b]).wait()
                gather(b)
        return 0
    lax.fori_loop(0, BLOCKS_PER_TILE // NBUF, group, 0)

    pltpu.make_async_copy(rowsf, accum.at[dst_all.at[BLOCKS_PER_TILE - 1]],
                          ssems[0]).wait()
    plsc.subcore_barrier()
    pltpu.sync_copy(accum.at[pl.ds(row0, ROWS_PER_TILE)],
                    out_hbm.at[c, pl.ds(row0, ROWS_PER_TILE)])


def _sc_propagate(table_words, srcs, dsts, zeros_rows):
    f = functools.partial(
        pl.kernel,
        out_type=jax.ShapeDtypeStruct((2, R, L), jnp.float32),
        mesh=_mesh(),
        scratch_types=[
            pltpu.VMEM((NBUF, BSZ), jnp.int32),
            pltpu.VMEM((BLOCKS_PER_TILE, BSZ), jnp.int32),
            pltpu.VMEM((NBUF, BSZ, WPR), jnp.int32),
            pltpu.VMEM((BSZ, L), jnp.float32),
            pltpu.VMEM_SHARED((R, L), jnp.float32),
        ] + [pltpu.SemaphoreType.DMA] * (3 * NBUF),
        compiler_params=pltpu.CompilerParams(needs_layout_passes=False,
                                             use_tc_tiling_on_sc=False),
    )(_prop_body)
    return f(table_words, srcs, dsts, zeros_rows)


def _pack_bf16_words(x):
    b = jax.lax.convert_element_type(x, jnp.bfloat16)
    u = jax.lax.bitcast_convert_type(b, jnp.uint16).astype(jnp.uint32)
    g = u.reshape(-1, L // 32, 2, 16)
    w = g[:, :, 0, :] | (g[:, :, 1, :] << 16)
    return jax.lax.bitcast_convert_type(w.reshape(-1, WPR), jnp.int32)


def _mm_body(x_ref, w_ref, o_ref):
    o_ref[...] = jnp.dot(x_ref[...], w_ref[...], preferred_element_type=jnp.float32)


def _tc_matmul(x, w):
    return pl.pallas_call(
        _mm_body,
        grid=(NB,),
        in_specs=[
            pl.BlockSpec((BM, F), lambda i: (i, 0)),
            pl.BlockSpec((F, H), lambda i: (0, 0)),
        ],
        out_specs=pl.BlockSpec((BM, H), lambda i: (i, 0)),
        out_shape=jax.ShapeDtypeStruct((R, H), jnp.float32),
    )(x, w)


def _scale_body(u_ref, da_ref, db_ref, us_ref, dinv_ref):
    i = pl.program_id(0)
    rows = lax.broadcasted_iota(jnp.int32, (BM, 1), 0) + i * BM
    mask = (rows < N).astype(jnp.float32)
    dinv = lax.rsqrt(da_ref[...] + db_ref[...] + 1.0)
    dinv_ref[...] = dinv
    md = mask * dinv
    for c in range(2):
        us_ref[c] = md * u_ref[:, c * L:(c + 1) * L]


def _tc_scale(u, deg_a, deg_b):
    return pl.pallas_call(
        _scale_body,
        grid=(NB,),
        in_specs=[
            pl.BlockSpec((BM, H), lambda i: (i, 0)),
            pl.BlockSpec((BM, 1), lambda i: (i, 0)),
            pl.BlockSpec((BM, 1), lambda i: (i, 0)),
        ],
        out_specs=[
            pl.BlockSpec((2, BM, L), lambda i: (0, i, 0)),
            pl.BlockSpec((BM, 1), lambda i: (i, 0)),
        ],
        out_shape=[
            jax.ShapeDtypeStruct((2, R, L), jnp.float32),
            jax.ShapeDtypeStruct((R, 1), jnp.float32),
        ],
    )(u, deg_a, deg_b)


def _layer1_body(s1_ref, us_ref, dinv_ref, b1_ref, hs_ref):
    i = pl.program_id(0)
    rows = lax.broadcasted_iota(jnp.int32, (BM, 1), 0) + i * BM
    mask = (rows < N).astype(jnp.float32)
    dinv = dinv_ref[...]
    md = mask * dinv
    for c in range(2):
        t = dinv * (s1_ref[c] + us_ref[c]) + b1_ref[c]
        hs_ref[c] = md * jnp.tanh(t)


def _tc_layer1(s1, us2, dinv, b1_2):
    spec_c = pl.BlockSpec((2, BM, L), lambda i: (0, i, 0))
    return pl.pallas_call(
        _layer1_body,
        grid=(NB,),
        in_specs=[
            spec_c,
            spec_c,
            pl.BlockSpec((BM, 1), lambda i: (i, 0)),
            pl.BlockSpec((2, 1, L), lambda i: (0, 0, 0)),
        ],
        out_specs=spec_c,
        out_shape=jax.ShapeDtypeStruct((2, R, L), jnp.float32),
    )(s1, us2, dinv, b1_2)


def _head_body(s2_ref, hs_ref, dinv_ref, wm_ref, wl_ref, bm_ref, bl_ref,
               noise_ref, z_ref, mean_ref, lv_ref):
    dinv = dinv_ref[...]
    mean = bm_ref[...]
    lv = bl_ref[...]
    for c in range(2):
        pc = dinv * (s2_ref[c] + hs_ref[c])
        mean = mean + jnp.dot(pc, wm_ref[c], preferred_element_type=jnp.float32)
        lv = lv + jnp.dot(pc, wl_ref[c], preferred_element_type=jnp.float32)
    mean_ref[...] = mean
    lv_ref[...] = lv
    z_ref[...] = noise_ref[...] * jnp.exp(0.5 * lv) + mean


def _tc_head(s2, hs2, dinv, wm2, wl2, bm_2, bl_2, noise):
    spec_c = pl.BlockSpec((2, BM, L), lambda i: (0, i, 0))
    return pl.pallas_call(
        _head_body,
        grid=(NB,),
        in_specs=[
            spec_c,
            spec_c,
            pl.BlockSpec((BM, 1), lambda i: (i, 0)),
            pl.BlockSpec((2, L, L), lambda i: (0, 0, 0)),
            pl.BlockSpec((2, L, L), lambda i: (0, 0, 0)),
            pl.BlockSpec((1, L), lambda i: (0, 0)),
            pl.BlockSpec((1, L), lambda i: (0, 0)),
            pl.BlockSpec((BM, L), lambda i: (i, 0)),
        ],
        out_specs=[
            pl.BlockSpec((BM, L), lambda i: (i, 0)),
            pl.BlockSpec((BM, L), lambda i: (i, 0)),
            pl.BlockSpec((BM, L), lambda i: (i, 0)),
        ],
        out_shape=[
            jax.ShapeDtypeStruct((R, L), jnp.float32),
            jax.ShapeDtypeStruct((R, L), jnp.float32),
            jax.ShapeDtypeStruct((R, L), jnp.float32),
        ],
    )(s2, hs2, dinv, wm2, wl2, bm_2, bl_2, noise)


@jax.jit
def _run(feature, edge_index, W1, b1, Wm, bm, Wl, bl):
    src = edge_index[0]
    dst = edge_index[1]
    pad = jnp.full((EP - E,), N, dtype=jnp.int32)
    src_p = jnp.concatenate([src, pad])
    dst_p = jnp.concatenate([dst, pad])
    srcs = jnp.stack([src_p, src_p + R]).reshape(2, 16, BLOCKS_PER_TILE, BSZ)
    dsts_blocked = dst_p.reshape(16, BLOCKS_PER_TILE, BSZ)

    xp = jnp.zeros((R, F), jnp.float32).at[:N].set(feature)
    zeros_rows = jnp.zeros((R, L), jnp.float32)
    noise = jax.random.normal(jax.random.key(42), (N, L), dtype=jnp.float32)
    noise_p = jnp.zeros((R, L), jnp.float32).at[:N].set(noise)

    deg2 = _sc_degree(dst_p)
    deg_a = deg2[0].reshape(R, 1)
    deg_b = deg2[1].reshape(R, 1)

    u = _tc_matmul(xp, W1)
    us2, dinv = _tc_scale(u, deg_a, deg_b)

    s1 = _sc_propagate(_pack_bf16_words(us2.reshape(2 * R, L)), srcs,
                       dsts_blocked, zeros_rows)
    hs2 = _tc_layer1(s1, us2, dinv, b1.reshape(2, 1, L))
    s2 = _sc_propagate(_pack_bf16_words(hs2.reshape(2 * R, L)), srcs,
                       dsts_blocked, zeros_rows)

    z, mean, lv = _tc_head(s2, hs2, dinv, Wm.reshape(2, L, L), Wl.reshape(2, L, L),
                           bm.reshape(1, L), bl.reshape(1, L), noise_p)
    return z[:N], mean[:N], lv[:N]


def kernel(feature, edge_index, W1, b1, Wm, bm, Wl, bl):
    return _run(feature, edge_index, W1, b1, Wm, bm, Wl, bl)


# speedup vs baseline: 1.0757x; 1.0757x over previous
import functools

import jax
import jax.numpy as jnp
from jax import lax
from jax.experimental import pallas as pl
from jax.experimental.pallas import tpu as pltpu
from jax.experimental.pallas import tpu_sc as plsc

N = 10000
F = 256
H = 256
L = 128
E = 160000

R = 10240
EP = 163840
EDGES_PER_TILE = EP // 16
BSZ = 64
BLOCKS_PER_TILE = EDGES_PER_TILE // BSZ
DEG_CHUNK = EP // 32
ROWS_PER_TILE = R // 16
BM = 1024
NB = R // BM
NBUF = 2


def _mesh():
    return plsc.VectorSubcoreMesh(core_axis_name="c", subcore_axis_name="s")


def _deg_body(dsts_hbm, out_hbm, dst_v, hist_v, tmp_v, acc_v, shared):
    c = lax.axis_index("c")
    s = lax.axis_index("s")
    w = c * 16 + s
    z16 = jnp.zeros((16,), jnp.float32)

    def zero_hist(i, _):
        hist_v[pl.ds(i * 16, 16)] = z16
        return 0
    lax.fori_loop(0, R // 16, zero_hist, 0)

    pltpu.sync_copy(dsts_hbm.at[pl.ds(w * DEG_CHUNK, DEG_CHUNK)], dst_v)
    ones = jnp.ones((16,), jnp.float32)

    def hist_step(i, _):
        idx = dst_v[pl.ds(i * 16, 16)]
        plsc.addupdate_scatter(hist_v, [idx], ones)
        return 0
    lax.fori_loop(0, DEG_CHUNK // 16, hist_step, 0)

    pltpu.sync_copy(hist_v, shared.at[s])
    plsc.subcore_barrier()

    def zero_acc(i, _):
        acc_v[pl.ds(i * 16, 16)] = z16
        return 0
    lax.fori_loop(0, ROWS_PER_TILE // 16, zero_acc, 0)
    for t in range(16):
        pltpu.sync_copy(shared.at[t, pl.ds(s * ROWS_PER_TILE, ROWS_PER_TILE)], tmp_v)

        def add_step(j, _):
            acc_v[pl.ds(j * 16, 16)] = acc_v[pl.ds(j * 16, 16)] + tmp_v[pl.ds(j * 16, 16)]
            return 0
        lax.fori_loop(0, ROWS_PER_TILE // 16, add_step, 0)
    pltpu.sync_copy(acc_v, out_hbm.at[c, pl.ds(s * ROWS_PER_TILE, ROWS_PER_TILE)])


def _sc_degree(dsts):
    f = functools.partial(
        pl.kernel,
        out_type=jax.ShapeDtypeStruct((2, R), jnp.float32),
        mesh=_mesh(),
        scratch_types=[
            pltpu.VMEM((DEG_CHUNK,), jnp.int32),
            pltpu.VMEM((R,), jnp.float32),
            pltpu.VMEM((ROWS_PER_TILE,), jnp.float32),
            pltpu.VMEM((ROWS_PER_TILE,), jnp.float32),
            pltpu.VMEM_SHARED((16, R), jnp.float32),
        ],
        compiler_params=pltpu.CompilerParams(needs_layout_passes=False),
    )(_deg_body)
    return f(dsts)


WPR = L // 2

def _prop_body(table_hbm, srcs_hbm, dsts_hbm, zeros_hbm, out_hbm,
               sidx, dst_all, words, rowsf, accum, *sems):
    gsems = sems[:NBUF]
    ssems = sems[NBUF:2 * NBUF]
    isems = sems[2 * NBUF:]
    c = lax.axis_index("c")
    s = lax.axis_index("s")
    row0 = s * ROWS_PER_TILE
    pltpu.sync_copy(zeros_hbm.at[pl.ds(row0, ROWS_PER_TILE)],
                    accum.at[pl.ds(row0, ROWS_PER_TILE)])
    pltpu.sync_copy(dsts_hbm.at[s], dst_all)
    for b in range(NBUF):
        pltpu.sync_copy(srcs_hbm.at[c, s, b], sidx.at[b])
    plsc.subcore_barrier()

    def gather(b):
        return pltpu.async_copy(table_hbm.at[sidx.at[b]], words.at[b],
                                gsems[b])

    def scatter(k, b):
        return pltpu.async_copy(rowsf.at[b], accum.at[dst_all.at[k]],
                                ssems[b], add=True)

    mask_hi = jnp.full((16,), -65536, jnp.int32)

    def widen(b):
        def step(r0, _):
            for dr in range(8):
                r = r0 * 8 + dr
                for wv in range(WPR // 16):
                    w = words[b, r, pl.ds(wv * 16, 16)]
                    lo = plsc.bitcast(w << 16, jnp.float32)
                    hi = plsc.bitcast(w & mask_hi, jnp.float32)
                    rowsf[b, r, pl.ds(wv * 32, 16)] = lo
                    rowsf[b, r, pl.ds(wv * 32 + 16, 16)] = hi
            return 0
        lax.fori_loop(0, BSZ // 8, step, 0)

    for b in range(NBUF):
        gather(b)

    def group(k0, _):
        not_last = k0 < BLOCKS_PER_TILE // NBUF - 1
        for b in range(NBUF):
            k = k0 * NBUF + b
            pltpu.make_async_copy(table_hbm.at[sidx.at[b]], words.at[b],
                                  gsems[b]).wait()

            @pl.when(not_last)
            def _():
                pltpu.async_copy(srcs_hbm.at[c, s, k + NBUF], sidx.at[b],
                                 isems[b])

            @pl.when(k0 > 0)
            def _():
                pltpu.make_async_copy(rowsf.at[b], accum.at[dst_all.at[k]],
                                      ssems[b]).wait()
            widen(b)
            scatter(k, b)

            @pl.when(not_last)
            def _():
                pltpu.make_async_copy(srcs_hbm.at[c, s, k + NBUF], sidx.at[b],
                                      isems[b]).wait()
                gather(b)
        return 0
    lax.fori_loop(0, BLOCKS_PER_TILE // NBUF, group, 0)

    for b in range(NBUF):
        k_last = BLOCKS_PER_TILE - NBUF + b
        pltpu.make_async_copy(rowsf.at[b], accum.at[dst_all.at[k_last]],
                              ssems[b]).wait()
    plsc.subcore_barrier()
    pltpu.sync_copy(accum.at[pl.ds(row0, ROWS_PER_TILE)],
                    out_hbm.at[c, pl.ds(row0, ROWS_PER_TILE)])


def _sc_propagate(table_words, srcs, dsts, zeros_rows):
    f = functools.partial(
        pl.kernel,
        out_type=jax.ShapeDtypeStruct((2, R, L), jnp.float32),
        mesh=_mesh(),
        scratch_types=[
            pltpu.VMEM((NBUF, BSZ), jnp.int32),
            pltpu.VMEM((BLOCKS_PER_TILE, BSZ), jnp.int32),
            pltpu.VMEM((NBUF, BSZ, WPR), jnp.int32),
            pltpu.VMEM((NBUF, BSZ, L), jnp.float32),
            pltpu.VMEM_SHARED((R, L), jnp.float32),
        ] + [pltpu.SemaphoreType.DMA] * (3 * NBUF),
        compiler_params=pltpu.CompilerParams(needs_layout_passes=False,
                                             use_tc_tiling_on_sc=False),
    )(_prop_body)
    return f(table_words, srcs, dsts, zeros_rows)


def _pack_bf16_words(x):
    b = jax.lax.convert_element_type(x, jnp.bfloat16)
    u = jax.lax.bitcast_convert_type(b, jnp.uint16).astype(jnp.uint32)
    g = u.reshape(-1, L // 32, 2, 16)
    w = g[:, :, 0, :] | (g[:, :, 1, :] << 16)
    return jax.lax.bitcast_convert_type(w.reshape(-1, WPR), jnp.int32)


def _mm_body(x_ref, w_ref, o_ref):
    o_ref[...] = jnp.dot(x_ref[...], w_ref[...], preferred_element_type=jnp.float32)


def _tc_matmul(x, w):
    return pl.pallas_call(
        _mm_body,
        grid=(NB,),
        in_specs=[
            pl.BlockSpec((BM, F), lambda i: (i, 0)),
            pl.BlockSpec((F, H), lambda i: (0, 0)),
        ],
        out_specs=pl.BlockSpec((BM, H), lambda i: (i, 0)),
        out_shape=jax.ShapeDtypeStruct((R, H), jnp.float32),
    )(x, w)


def _scale_body(u_ref, da_ref, db_ref, us_ref, dinv_ref):
    i = pl.program_id(0)
    rows = lax.broadcasted_iota(jnp.int32, (BM, 1), 0) + i * BM
    mask = (rows < N).astype(jnp.float32)
    dinv = lax.rsqrt(da_ref[...] + db_ref[...] + 1.0)
    dinv_ref[...] = dinv
    md = mask * dinv
    for c in range(2):
        us_ref[c] = md * u_ref[:, c * L:(c + 1) * L]


def _tc_scale(u, deg_a, deg_b):
    return pl.pallas_call(
        _scale_body,
        grid=(NB,),
        in_specs=[
            pl.BlockSpec((BM, H), lambda i: (i, 0)),
            pl.BlockSpec((BM, 1), lambda i: (i, 0)),
            pl.BlockSpec((BM, 1), lambda i: (i, 0)),
        ],
        out_specs=[
            pl.BlockSpec((2, BM, L), lambda i: (0, i, 0)),
            pl.BlockSpec((BM, 1), lambda i: (i, 0)),
        ],
        out_shape=[
            jax.ShapeDtypeStruct((2, R, L), jnp.float32),
            jax.ShapeDtypeStruct((R, 1), jnp.float32),
        ],
    )(u, deg_a, deg_b)


def _layer1_body(s1_ref, us_ref, dinv_ref, b1_ref, hs_ref):
    i = pl.program_id(0)
    rows = lax.broadcasted_iota(jnp.int32, (BM, 1), 0) + i * BM
    mask = (rows < N).astype(jnp.float32)
    dinv = dinv_ref[...]
    md = mask * dinv
    for c in range(2):
        t = dinv * (s1_ref[c] + us_ref[c]) + b1_ref[c]
        hs_ref[c] = md * jnp.tanh(t)


def _tc_layer1(s1, us2, dinv, b1_2):
    spec_c = pl.BlockSpec((2, BM, L), lambda i: (0, i, 0))
    return pl.pallas_call(
        _layer1_body,
        grid=(NB,),
        in_specs=[
            spec_c,
            spec_c,
            pl.BlockSpec((BM, 1), lambda i: (i, 0)),
            pl.BlockSpec((2, 1, L), lambda i: (0, 0, 0)),
        ],
        out_specs=spec_c,
        out_shape=jax.ShapeDtypeStruct((2, R, L), jnp.float32),
    )(s1, us2, dinv, b1_2)


def _head_body(s2_ref, hs_ref, dinv_ref, wm_ref, wl_ref, bm_ref, bl_ref,
               noise_ref, z_ref, mean_ref, lv_ref):
    dinv = dinv_ref[...]
    mean = bm_ref[...]
    lv = bl_ref[...]
    for c in range(2):
        pc = dinv * (s2_ref[c] + hs_ref[c])
        mean = mean + jnp.dot(pc, wm_ref[c], preferred_element_type=jnp.float32)
        lv = lv + jnp.dot(pc, wl_ref[c], preferred_element_type=jnp.float32)
    mean_ref[...] = mean
    lv_ref[...] = lv
    z_ref[...] = noise_ref[...] * jnp.exp(0.5 * lv) + mean


def _tc_head(s2, hs2, dinv, wm2, wl2, bm_2, bl_2, noise):
    spec_c = pl.BlockSpec((2, BM, L), lambda i: (0, i, 0))
    return pl.pallas_call(
        _head_body,
        grid=(NB,),
        in_specs=[
            spec_c,
            spec_c,
            pl.BlockSpec((BM, 1), lambda i: (i, 0)),
            pl.BlockSpec((2, L, L), lambda i: (0, 0, 0)),
            pl.BlockSpec((2, L, L), lambda i: (0, 0, 0)),
            pl.BlockSpec((1, L), lambda i: (0, 0)),
            pl.BlockSpec((1, L), lambda i: (0, 0)),
            pl.BlockSpec((BM, L), lambda i: (i, 0)),
        ],
        out_specs=[
            pl.BlockSpec((BM, L), lambda i: (i, 0)),
            pl.BlockSpec((BM, L), lambda i: (i, 0)),
            pl.BlockSpec((BM, L), lambda i: (i, 0)),
        ],
        out_shape=[
            jax.ShapeDtypeStruct((R, L), jnp.float32),
            jax.ShapeDtypeStruct((R, L), jnp.float32),
            jax.ShapeDtypeStruct((R, L), jnp.float32),
        ],
    )(s2, hs2, dinv, wm2, wl2, bm_2, bl_2, noise)


@jax.jit
def _run(feature, edge_index, W1, b1, Wm, bm, Wl, bl):
    src = edge_index[0]
    dst = edge_index[1]
    pad = jnp.full((EP - E,), N, dtype=jnp.int32)
    src_p = jnp.concatenate([src, pad])
    dst_p = jnp.concatenate([dst, pad])
    srcs = jnp.stack([src_p, src_p + R]).reshape(2, 16, BLOCKS_PER_TILE, BSZ)
    dsts_blocked = dst_p.reshape(16, BLOCKS_PER_TILE, BSZ)

    xp = jnp.zeros((R, F), jnp.float32).at[:N].set(feature)
    zeros_rows = jnp.zeros((R, L), jnp.float32)
    noise = jax.random.normal(jax.random.key(42), (N, L), dtype=jnp.float32)
    noise_p = jnp.zeros((R, L), jnp.float32).at[:N].set(noise)

    deg2 = _sc_degree(dst_p)
    deg_a = deg2[0].reshape(R, 1)
    deg_b = deg2[1].reshape(R, 1)

    u = _tc_matmul(xp, W1)
    us2, dinv = _tc_scale(u, deg_a, deg_b)

    s1 = _sc_propagate(_pack_bf16_words(us2.reshape(2 * R, L)), srcs,
                       dsts_blocked, zeros_rows)
    hs2 = _tc_layer1(s1, us2, dinv, b1.reshape(2, 1, L))
    s2 = _sc_propagate(_pack_bf16_words(hs2.reshape(2 * R, L)), srcs,
                       dsts_blocked, zeros_rows)

    z, mean, lv = _tc_head(s2, hs2, dinv, Wm.reshape(2, L, L), Wl.reshape(2, L, L),
                           bm.reshape(1, L), bl.reshape(1, L), noise_p)
    return z[:N], mean[:N], lv[:N]


def kernel(feature, edge_index, W1, b1, Wm, bm, Wl, bl):
    return _run(feature, edge_index, W1, b1, Wm, bm, Wl, bl)
